# baseline (speedup 1.0000x reference)
import functools

import jax
import jax.numpy as jnp
from jax import lax
from jax.experimental import pallas as pl
from jax.experimental.pallas import tpu as pltpu
from jax.experimental.pallas import tpu_sc as plsc

MAX_VISITS = 512
EMBED_DIM = 64
BATCH = 16384
SEQ = 200

_info = plsc.get_sparse_core_info()
NC, NS = _info.num_cores, _info.num_subcores
NW = NC * NS
B_PER_W = BATCH // NW
NJ = B_PER_W // 16

_mesh = plsc.VectorSubcoreMesh(core_axis_name="c", subcore_axis_name="s")


@functools.partial(
    pl.kernel,
    mesh=_mesh,
    out_type=jax.ShapeDtypeStruct((SEQ, EMBED_DIM, BATCH), jnp.float32),
    scratch_types=[
        pltpu.VMEM((MAX_VISITS * EMBED_DIM,), jnp.float32),
        pltpu.VMEM((B_PER_W,), jnp.int32),
        pltpu.VMEM((B_PER_W,), jnp.int32),
        pltpu.VMEM((EMBED_DIM, B_PER_W), jnp.float32),
        pltpu.VMEM((EMBED_DIM, B_PER_W), jnp.float32),
        pltpu.SemaphoreType.DMA,
        pltpu.SemaphoreType.DMA,
        pltpu.SemaphoreType.DMA,
        pltpu.SemaphoreType.DMA,
    ],
    compiler_params=pltpu.CompilerParams(needs_layout_passes=False),
)
def _embed_kernel(ids_t_hbm, tab_t_hbm, out_hbm, tab_v, ids0, ids1,
                  out0, out1, isem0, isem1, osem0, osem1):
    wid = lax.axis_index("c") * NS + lax.axis_index("s")
    b0 = wid * B_PER_W

    ids_bufs = (ids0, ids1)
    out_bufs = (out0, out1)
    isems = (isem0, isem1)
    osems = (osem0, osem1)

    pltpu.sync_copy(tab_t_hbm, tab_v)

    pltpu.async_copy(ids_t_hbm.at[0, pl.ds(b0, B_PER_W)], ids0, isem0)
    pltpu.async_copy(ids_t_hbm.at[1, pl.ds(b0, B_PER_W)], ids1, isem1)

    def step(i, _):
        for half in (0, 1):
            s = 2 * i + half
            ids_v = ids_bufs[half]
            out_v = out_bufs[half]
            pltpu.make_async_copy(
                ids_t_hbm.at[0, pl.ds(b0, B_PER_W)], ids_v, isems[half]
            ).wait()
            @pl.when(s >= 2)
            def _():
                pltpu.make_async_copy(
                    out_v, out_hbm.at[0, :, pl.ds(b0, B_PER_W)], osems[half]
                ).wait()

            @plsc.parallel_loop(0, NJ, step=1, unroll=4)
            def _gather_group(j):
                idvec = ids_v[pl.ds(j * 16, 16)]
                for e in range(EMBED_DIM):
                    out_v[e, pl.ds(j * 16, 16)] = plsc.load_gather(
                        tab_v, [idvec + (e * MAX_VISITS)]
                    )

            @pl.when(s + 2 < SEQ)
            def _():
                pltpu.async_copy(
                    ids_t_hbm.at[s + 2, pl.ds(b0, B_PER_W)], ids_v, isems[half]
                )

            pltpu.async_copy(
                out_v, out_hbm.at[s, :, pl.ds(b0, B_PER_W)], osems[half]
            )
        return ()

    lax.fori_loop(0, SEQ // 2, step, ())

    for half in (0, 1):
        pltpu.make_async_copy(
            out_bufs[half], out_hbm.at[0, :, pl.ds(b0, B_PER_W)], osems[half]
        ).wait()


def kernel(visit_ids, table):
    ids_t = visit_ids.T.astype(jnp.int32)
    tab_t = table.T.reshape(EMBED_DIM * MAX_VISITS)
    out = _embed_kernel(ids_t, tab_t)
    return out.transpose(2, 0, 1)


# speedup vs baseline: 1.1484x; 1.1484x over previous
import functools

import jax
import jax.numpy as jnp
from jax import lax
from jax.experimental import pallas as pl
from jax.experimental.pallas import tpu as pltpu
from jax.experimental.pallas import tpu_sc as plsc

MAX_VISITS = 512
EMBED_DIM = 64
BATCH = 16384
SEQ = 200

_info = plsc.get_sparse_core_info()
NC, NS = _info.num_cores, _info.num_subcores
NW = NC * NS
B_PER_W = BATCH // NW
NJ = B_PER_W // 16

_mesh = plsc.VectorSubcoreMesh(core_axis_name="c", subcore_axis_name="s")


@functools.partial(
    pl.kernel,
    mesh=_mesh,
    out_type=jax.ShapeDtypeStruct((SEQ, EMBED_DIM, BATCH), jnp.float32),
    scratch_types=[
        pltpu.VMEM((MAX_VISITS * EMBED_DIM,), jnp.float32),
        pltpu.VMEM((B_PER_W,), jnp.int32),
        pltpu.VMEM((B_PER_W,), jnp.int32),
        pltpu.VMEM((EMBED_DIM, B_PER_W), jnp.float32),
        pltpu.VMEM((EMBED_DIM, B_PER_W), jnp.float32),
        pltpu.SemaphoreType.DMA,
        pltpu.SemaphoreType.DMA,
        pltpu.SemaphoreType.DMA,
        pltpu.SemaphoreType.DMA,
    ],
    compiler_params=pltpu.CompilerParams(needs_layout_passes=False),
)
def _embed_kernel(ids_t_hbm, tab_t_hbm, out_hbm, tab_v, ids0, ids1,
                  out0, out1, isem0, isem1, osem0, osem1):
    wid = lax.axis_index("c") * NS + lax.axis_index("s")
    b0 = wid * B_PER_W

    ids_bufs = (ids0, ids1)
    out_bufs = (out0, out1)
    isems = (isem0, isem1)
    osems = (osem0, osem1)

    pltpu.sync_copy(tab_t_hbm, tab_v)

    pltpu.async_copy(ids_t_hbm.at[0, pl.ds(b0, B_PER_W)], ids0, isem0)
    pltpu.async_copy(ids_t_hbm.at[1, pl.ds(b0, B_PER_W)], ids1, isem1)

    def step(i, _):
        for half in (0, 1):
            s = 2 * i + half
            ids_v = ids_bufs[half]
            out_v = out_bufs[half]
            pltpu.make_async_copy(
                ids_t_hbm.at[0, pl.ds(b0, B_PER_W)], ids_v, isems[half]
            ).wait()
            @pl.when(s >= 2)
            def _():
                pltpu.make_async_copy(
                    out_v, out_hbm.at[0, :, pl.ds(b0, B_PER_W)], osems[half]
                ).wait()

            @plsc.parallel_loop(0, NJ, step=1, unroll=2)
            def _gather_group(j):
                idvec = ids_v[pl.ds(j * 16, 16)]
                for e in range(EMBED_DIM):
                    out_v[e, pl.ds(j * 16, 16)] = plsc.load_gather(
                        tab_v, [idvec + (e * MAX_VISITS)]
                    )

            @pl.when(s + 2 < SEQ)
            def _():
                pltpu.async_copy(
                    ids_t_hbm.at[s + 2, pl.ds(b0, B_PER_W)], ids_v, isems[half]
                )

            pltpu.async_copy(
                out_v, out_hbm.at[s, :, pl.ds(b0, B_PER_W)], osems[half]
            )
        return ()

    lax.fori_loop(0, SEQ // 2, step, ())

    for half in (0, 1):
        pltpu.make_async_copy(
            out_bufs[half], out_hbm.at[0, :, pl.ds(b0, B_PER_W)], osems[half]
        ).wait()


def kernel(visit_ids, table):
    ids_t = visit_ids.T.astype(jnp.int32)
    tab_t = table.T.reshape(EMBED_DIM * MAX_VISITS)
    out = _embed_kernel(ids_t, tab_t)
    return out.transpose(2, 0, 1)


# speedup vs baseline: 1.4011x; 1.2200x over previous
import functools

import jax
import jax.numpy as jnp
from jax import lax
from jax.experimental import pallas as pl
from jax.experimental.pallas import tpu as pltpu
from jax.experimental.pallas import tpu_sc as plsc

MAX_VISITS = 512
EMBED_DIM = 64
BATCH = 16384
SEQ = 200

_info = plsc.get_sparse_core_info()
NC, NS = _info.num_cores, _info.num_subcores
NW = NC * NS
N_STRIPES = 16
B_PER_W = BATCH // N_STRIPES
S_PER_W = SEQ // (NW // N_STRIPES)
E_HALF = EMBED_DIM // 2
NJ = B_PER_W // 16

_mesh = plsc.VectorSubcoreMesh(core_axis_name="c", subcore_axis_name="s")


@functools.partial(
    pl.kernel,
    mesh=_mesh,
    out_type=jax.ShapeDtypeStruct((SEQ, EMBED_DIM, BATCH), jnp.float32),
    scratch_types=[
        pltpu.VMEM((MAX_VISITS * EMBED_DIM,), jnp.float32),
        pltpu.VMEM((B_PER_W,), jnp.int32),
        pltpu.VMEM((B_PER_W,), jnp.int32),
        pltpu.VMEM((E_HALF, B_PER_W), jnp.float32),
        pltpu.VMEM((E_HALF, B_PER_W), jnp.float32),
        pltpu.SemaphoreType.DMA,
        pltpu.SemaphoreType.DMA,
        pltpu.SemaphoreType.DMA,
        pltpu.SemaphoreType.DMA,
    ],
    compiler_params=pltpu.CompilerParams(needs_layout_passes=False),
)
def _embed_kernel(ids_t_hbm, tab_t_hbm, out_hbm, tab_v, ids0, ids1,
                  out0, out1, isem0, isem1, osem0, osem1):
    wid = lax.axis_index("c") * NS + lax.axis_index("s")
    stripe = lax.rem(wid, N_STRIPES)
    shalf = wid // N_STRIPES
    b0 = stripe * B_PER_W
    s_base = shalf * S_PER_W

    ids_bufs = (ids0, ids1)
    out_bufs = (out0, out1)
    isems = (isem0, isem1)
    osems = (osem0, osem1)

    pltpu.sync_copy(tab_t_hbm, tab_v)

    pltpu.async_copy(ids_t_hbm.at[s_base, pl.ds(b0, B_PER_W)], ids0, isem0)
    pltpu.async_copy(ids_t_hbm.at[s_base + 1, pl.ds(b0, B_PER_W)], ids1, isem1)

    def pair(i, _):
        for ih in (0, 1):
            sl = 2 * i + ih
            s = s_base + sl
            ids_v = ids_bufs[ih]
            pltpu.make_async_copy(
                ids_t_hbm.at[0, pl.ds(b0, B_PER_W)], ids_v, isems[ih]
            ).wait()
            for eh in (0, 1):
                out_v = out_bufs[eh]
                u = 2 * sl + eh
                @pl.when(u >= 2)
                def _():
                    pltpu.make_async_copy(
                        out_v,
                        out_hbm.at[0, pl.ds(0, E_HALF), pl.ds(b0, B_PER_W)],
                        osems[eh],
                    ).wait()

                @plsc.parallel_loop(0, NJ, step=1, unroll=2)
                def _gather_group(j):
                    idvec = ids_v[pl.ds(j * 16, 16)]
                    for e in range(E_HALF):
                        out_v[e, pl.ds(j * 16, 16)] = plsc.load_gather(
                            tab_v, [idvec + ((eh * E_HALF + e) * MAX_VISITS)]
                        )

                pltpu.async_copy(
                    out_v,
                    out_hbm.at[s, pl.ds(eh * E_HALF, E_HALF), pl.ds(b0, B_PER_W)],
                    osems[eh],
                )
            @pl.when(sl + 2 < S_PER_W)
            def _():
                pltpu.async_copy(
                    ids_t_hbm.at[s + 2, pl.ds(b0, B_PER_W)], ids_v, isems[ih]
                )
        return ()

    lax.fori_loop(0, S_PER_W // 2, pair, ())

    for eh in (0, 1):
        pltpu.make_async_copy(
            out_bufs[eh],
            out_hbm.at[0, pl.ds(0, E_HALF), pl.ds(b0, B_PER_W)],
            osems[eh],
        ).wait()


def kernel(visit_ids, table):
    ids_t = visit_ids.T.astype(jnp.int32)
    tab_t = table.T.reshape(EMBED_DIM * MAX_VISITS)
    out = _embed_kernel(ids_t, tab_t)
    return out.transpose(2, 0, 1)


# speedup vs baseline: 1.4579x; 1.0406x over previous
import functools

import jax
import jax.numpy as jnp
from jax import lax
from jax.experimental import pallas as pl
from jax.experimental.pallas import tpu as pltpu
from jax.experimental.pallas import tpu_sc as plsc

MAX_VISITS = 512
EMBED_DIM = 64
BATCH = 16384
SEQ = 200

_info = plsc.get_sparse_core_info()
NC, NS = _info.num_cores, _info.num_subcores
NW = NC * NS
N_STRIPES = 8
B_PER_W = BATCH // N_STRIPES
S_PER_W = SEQ * N_STRIPES // NW
E_BLK = (32 * 1024) // B_PER_W
NEB = EMBED_DIM // E_BLK
NJ = B_PER_W // 16

_mesh = plsc.VectorSubcoreMesh(core_axis_name="c", subcore_axis_name="s")


@functools.partial(
    pl.kernel,
    mesh=_mesh,
    out_type=jax.ShapeDtypeStruct((SEQ, EMBED_DIM, BATCH), jnp.float32),
    scratch_types=[
        pltpu.VMEM((MAX_VISITS * EMBED_DIM,), jnp.float32),
        pltpu.VMEM((B_PER_W,), jnp.int32),
        pltpu.VMEM((B_PER_W,), jnp.int32),
        pltpu.VMEM((E_BLK, B_PER_W), jnp.float32),
        pltpu.VMEM((E_BLK, B_PER_W), jnp.float32),
        pltpu.SemaphoreType.DMA,
        pltpu.SemaphoreType.DMA,
        pltpu.SemaphoreType.DMA,
        pltpu.SemaphoreType.DMA,
    ],
    compiler_params=pltpu.CompilerParams(needs_layout_passes=False),
)
def _embed_kernel(ids_t_hbm, tab_t_hbm, out_hbm, tab_v, ids0, ids1,
                  out0, out1, isem0, isem1, osem0, osem1):
    wid = lax.axis_index("c") * NS + lax.axis_index("s")
    stripe = lax.rem(wid, N_STRIPES)
    sblk = wid // N_STRIPES
    b0 = stripe * B_PER_W
    s_base = sblk * S_PER_W

    ids_bufs = (ids0, ids1)
    out_bufs = (out0, out1)
    isems = (isem0, isem1)
    osems = (osem0, osem1)

    pltpu.sync_copy(tab_t_hbm, tab_v)

    pltpu.async_copy(ids_t_hbm.at[s_base, pl.ds(b0, B_PER_W)], ids0, isem0)
    pltpu.async_copy(ids_t_hbm.at[s_base + 1, pl.ds(b0, B_PER_W)], ids1, isem1)

    def pair(i, _):
        for ih in (0, 1):
            sl = 2 * i + ih
            s = s_base + sl
            ids_v = ids_bufs[ih]
            pltpu.make_async_copy(
                ids_t_hbm.at[0, pl.ds(b0, B_PER_W)], ids_v, isems[ih]
            ).wait()
            for eb in range(NEB):
                out_v = out_bufs[eb % 2]
                u = NEB * sl + eb
                @pl.when(u >= 2)
                def _():
                    pltpu.make_async_copy(
                        out_v,
                        out_hbm.at[0, pl.ds(0, E_BLK), pl.ds(b0, B_PER_W)],
                        osems[eb % 2],
                    ).wait()

                @plsc.parallel_loop(0, NJ, step=1, unroll=2)
                def _gather_group(j):
                    idvec = ids_v[pl.ds(j * 16, 16)]
                    for e in range(E_BLK):
                        out_v[e, pl.ds(j * 16, 16)] = plsc.load_gather(
                            tab_v, [idvec + ((eb * E_BLK + e) * MAX_VISITS)]
                        )

                pltpu.async_copy(
                    out_v,
                    out_hbm.at[s, pl.ds(eb * E_BLK, E_BLK), pl.ds(b0, B_PER_W)],
                    osems[eb % 2],
                )
            @pl.when(sl + 2 < S_PER_W)
            def _():
                pltpu.async_copy(
                    ids_t_hbm.at[s + 2, pl.ds(b0, B_PER_W)], ids_v, isems[ih]
                )
        return ()

    lax.fori_loop(0, S_PER_W // 2, pair, ())

    for h in (0, 1):
        pltpu.make_async_copy(
            out_bufs[h],
            out_hbm.at[0, pl.ds(0, E_BLK), pl.ds(b0, B_PER_W)],
            osems[h],
        ).wait()


def kernel(visit_ids, table):
    ids_t = visit_ids.T.astype(jnp.int32)
    tab_t = table.T.reshape(EMBED_DIM * MAX_VISITS)
    out = _embed_kernel(ids_t, tab_t)
    return out.transpose(2, 0, 1)


# speedup vs baseline: 1.4887x; 1.0211x over previous
import functools

import jax
import jax.numpy as jnp
from jax import lax
from jax.experimental import pallas as pl
from jax.experimental.pallas import tpu as pltpu
from jax.experimental.pallas import tpu_sc as plsc

MAX_VISITS = 512
EMBED_DIM = 64
BATCH = 16384
SEQ = 200

_info = plsc.get_sparse_core_info()
NC, NS = _info.num_cores, _info.num_subcores
NW = NC * NS
N_STRIPES = 4
B_PER_W = BATCH // N_STRIPES
S_PER_W = SEQ * N_STRIPES // NW
E_BLK = (32 * 1024) // B_PER_W
NEB = EMBED_DIM // E_BLK
NJ = B_PER_W // 16

_mesh = plsc.VectorSubcoreMesh(core_axis_name="c", subcore_axis_name="s")


@functools.partial(
    pl.kernel,
    mesh=_mesh,
    out_type=jax.ShapeDtypeStruct((SEQ, EMBED_DIM, BATCH), jnp.float32),
    scratch_types=[
        pltpu.VMEM((MAX_VISITS * EMBED_DIM,), jnp.float32),
        pltpu.VMEM((B_PER_W,), jnp.int32),
        pltpu.VMEM((B_PER_W,), jnp.int32),
        pltpu.VMEM((E_BLK, B_PER_W), jnp.float32),
        pltpu.VMEM((E_BLK, B_PER_W), jnp.float32),
        pltpu.SemaphoreType.DMA,
        pltpu.SemaphoreType.DMA,
        pltpu.SemaphoreType.DMA,
        pltpu.SemaphoreType.DMA,
    ],
    compiler_params=pltpu.CompilerParams(needs_layout_passes=False),
)
def _embed_kernel(ids_t_hbm, tab_t_hbm, out_hbm, tab_v, ids0, ids1,
                  out0, out1, isem0, isem1, osem0, osem1):
    wid = lax.axis_index("c") * NS + lax.axis_index("s")
    stripe = lax.rem(wid, N_STRIPES)
    sblk = wid // N_STRIPES
    b0 = stripe * B_PER_W
    s_base = sblk * S_PER_W

    ids_bufs = (ids0, ids1)
    out_bufs = (out0, out1)
    isems = (isem0, isem1)
    osems = (osem0, osem1)

    pltpu.sync_copy(tab_t_hbm, tab_v)

    pltpu.async_copy(ids_t_hbm.at[s_base, pl.ds(b0, B_PER_W)], ids0, isem0)
    pltpu.async_copy(ids_t_hbm.at[s_base + 1, pl.ds(b0, B_PER_W)], ids1, isem1)

    def process_s(sl, ih):
        s = s_base + sl
        ids_v = ids_bufs[ih]
        pltpu.make_async_copy(
            ids_t_hbm.at[0, pl.ds(b0, B_PER_W)], ids_v, isems[ih]
        ).wait()
        for eb in range(NEB):
            out_v = out_bufs[eb % 2]
            u = NEB * sl + eb
            @pl.when(u >= 2)
            def _():
                pltpu.make_async_copy(
                    out_v,
                    out_hbm.at[0, pl.ds(0, E_BLK), pl.ds(b0, B_PER_W)],
                    osems[eb % 2],
                ).wait()

            @plsc.parallel_loop(0, NJ, step=1, unroll=2)
            def _gather_group(j):
                idvec = ids_v[pl.ds(j * 16, 16)]
                for e in range(E_BLK):
                    out_v[e, pl.ds(j * 16, 16)] = plsc.load_gather(
                        tab_v, [idvec + ((eb * E_BLK + e) * MAX_VISITS)]
                    )

            pltpu.async_copy(
                out_v,
                out_hbm.at[s, pl.ds(eb * E_BLK, E_BLK), pl.ds(b0, B_PER_W)],
                osems[eb % 2],
            )
        @pl.when(sl + 2 < S_PER_W)
        def _():
            pltpu.async_copy(
                ids_t_hbm.at[s + 2, pl.ds(b0, B_PER_W)], ids_v, isems[ih]
            )

    def pair(i, _):
        process_s(2 * i, 0)
        process_s(2 * i + 1, 1)
        return ()

    lax.fori_loop(0, S_PER_W // 2, pair, ())
    if S_PER_W % 2:
        process_s(S_PER_W - 1, (S_PER_W - 1) % 2)

    for h in (0, 1):
        pltpu.make_async_copy(
            out_bufs[h],
            out_hbm.at[0, pl.ds(0, E_BLK), pl.ds(b0, B_PER_W)],
            osems[h],
        ).wait()


def kernel(visit_ids, table):
    ids_t = visit_ids.T.astype(jnp.int32)
    tab_t = table.T.reshape(EMBED_DIM * MAX_VISITS)
    out = _embed_kernel(ids_t, tab_t)
    return out.transpose(2, 0, 1)
